# baseline (speedup 1.0000x reference)
import functools

import jax
import jax.numpy as jnp
from jax import lax
from jax.experimental import pallas as pl
from jax.experimental.pallas import tpu as pltpu

NUM_SEGMENTS = 2048
R = 1024
SB = 8


def _segment_reduce_body(wl_j, wl_chunk, wl_valid,
                         feats_ref, ids_ref, gw_ref, gb_ref,
                         sum_ref, gsum_ref, mx_ref, cnt_ref,
                         *, n_rows):
    t = pl.program_id(0)
    j = wl_j[t]
    chunk = wl_chunk[t]
    valid = wl_valid[t]

    h = feats_ref[...]
    ids = ids_ref[...]

    row_idx = chunk * R + lax.broadcasted_iota(jnp.int32, (R, 1), 0)
    seg_iota = lax.broadcasted_iota(jnp.int32, (1, SB), 1) + j * SB
    oh_bool = (ids == seg_iota) & (row_idx < n_rows) & (valid > 0)
    ohf = oh_bool.astype(jnp.float32)

    glog = jnp.dot(h, gw_ref[...], preferred_element_type=jnp.float32)
    g = jax.nn.sigmoid(glog[:, :1] + gb_ref[0, 0])

    ohcat = jnp.concatenate([ohf, ohf * g], axis=1)
    partial = lax.dot_general(ohcat, h, (((0,), (0,)), ((), ())),
                              preferred_element_type=jnp.float32)
    psum = partial[:SB, :]
    pgsum = partial[SB:, :]
    pcnt = jnp.broadcast_to(jnp.sum(ohf, axis=0).reshape(SB, 1), (SB, 128))

    cols = []
    for s in range(SB):
        masked = jnp.where(oh_bool[:, s:s + 1], h, -jnp.inf)
        cols.append(jnp.max(masked, axis=0, keepdims=True))
    pmax = jnp.concatenate(cols, axis=0)

    prev_j = wl_j[jnp.maximum(t - 1, 0)]
    first = (t == 0) | (j != prev_j)

    @pl.when(first)
    def _():
        sum_ref[...] = psum
        gsum_ref[...] = pgsum
        mx_ref[...] = pmax
        cnt_ref[...] = pcnt

    @pl.when(jnp.logical_not(first))
    def _():
        sum_ref[...] += psum
        gsum_ref[...] += pgsum
        mx_ref[...] = jnp.maximum(mx_ref[...], pmax)
        cnt_ref[...] += pcnt


def _segment_reduce(feats, batch, gate_w, gate_b, num_segments):
    n, hdim = feats.shape
    nb = -(-n // R)
    npad = nb * R - n
    if npad:
        feats = jnp.pad(feats, ((0, npad), (0, 0)))
        batch = jnp.pad(batch, (0, npad))
    batch = batch.astype(jnp.int32)
    ids2 = batch.reshape(nb * R, 1)

    nj = num_segments // SB
    bounds = jnp.arange(0, num_segments + 1, SB, dtype=batch.dtype)
    jstarts = jnp.searchsorted(batch[:n] if npad else batch, bounds,
                               side='left').astype(jnp.int32)
    first_c = jnp.minimum(jstarts[:-1] // R, nb - 1)
    nonempty = jstarts[1:] > jstarts[:-1]
    last_c = jnp.where(nonempty, (jstarts[1:] - 1) // R, first_c)
    nch = last_c - first_c + 1
    off = jnp.concatenate([jnp.zeros((1,), jnp.int32), jnp.cumsum(nch)])
    total = off[-1]

    t_len = nb + nj
    t_idx = jnp.arange(t_len, dtype=jnp.int32)
    jj = jnp.clip(jnp.searchsorted(off, t_idx, side='right') - 1, 0, nj - 1)
    jj = jj.astype(jnp.int32)
    ch = jnp.clip(first_c[jj] + (t_idx - off[jj]), 0, nb - 1).astype(jnp.int32)
    wl_valid = (t_idx < total).astype(jnp.int32)

    gw = jnp.broadcast_to(gate_w.reshape(hdim, 1), (hdim, 128))
    gb = jnp.broadcast_to(gate_b.reshape(1, 1), (1, 128))

    grid_spec = pltpu.PrefetchScalarGridSpec(
        num_scalar_prefetch=3,
        grid=(t_len,),
        in_specs=[
            pl.BlockSpec((R, hdim), lambda t, wj, wc, wv: (wc[t], 0)),
            pl.BlockSpec((R, 1), lambda t, wj, wc, wv: (wc[t], 0)),
            pl.BlockSpec((hdim, 128), lambda t, wj, wc, wv: (0, 0)),
            pl.BlockSpec((1, 128), lambda t, wj, wc, wv: (0, 0)),
        ],
        out_specs=[
            pl.BlockSpec((SB, hdim), lambda t, wj, wc, wv: (wj[t], 0)),
            pl.BlockSpec((SB, hdim), lambda t, wj, wc, wv: (wj[t], 0)),
            pl.BlockSpec((SB, hdim), lambda t, wj, wc, wv: (wj[t], 0)),
            pl.BlockSpec((SB, 128), lambda t, wj, wc, wv: (wj[t], 0)),
        ],
    )
    out_shape = [
        jax.ShapeDtypeStruct((num_segments, hdim), jnp.float32),
        jax.ShapeDtypeStruct((num_segments, hdim), jnp.float32),
        jax.ShapeDtypeStruct((num_segments, hdim), jnp.float32),
        jax.ShapeDtypeStruct((num_segments, 128), jnp.float32),
    ]
    body = functools.partial(_segment_reduce_body, n_rows=n)
    return pl.pallas_call(
        body,
        grid_spec=grid_spec,
        out_shape=out_shape,
        compiler_params=pltpu.CompilerParams(
            dimension_semantics=("arbitrary",)),
    )(jj, ch, wl_valid, feats, ids2, gw, gb)


def _head_body(sum0_ref, gsum0_ref, mx0_ref, cnt0_ref,
               sum1_ref, gsum1_ref, mx1_ref, cnt1_ref,
               pw0_ref, pb0_ref, pw1_ref, pb1_ref,
               lng_ref, lnb_ref, f1w_ref, f1b_ref, f2w_ref, f2b_ref,
               out_ref):
    def rank(sum_ref, gsum_ref, mx_ref, cnt_ref, pw_ref, pb_ref):
        s = sum_ref[...]
        gs = gsum_ref[...]
        cnt = cnt_ref[:, :1]
        mean = s / jnp.maximum(cnt, 1.0)
        mx = jnp.where(cnt > 0, mx_ref[...], 0.0)
        agg = jnp.concatenate([s, mean, mx, gs], axis=1)
        return jnp.dot(agg, pw_ref[...],
                       preferred_element_type=jnp.float32) + pb_ref[...]

    r0 = rank(sum0_ref, gsum0_ref, mx0_ref, cnt0_ref, pw0_ref, pb0_ref)
    r1 = rank(sum1_ref, gsum1_ref, mx1_ref, cnt1_ref, pw1_ref, pb1_ref)
    state = jnp.concatenate([r0, r1], axis=1)
    mu = jnp.mean(state, axis=1, keepdims=True)
    var = jnp.mean((state - mu) ** 2, axis=1, keepdims=True)
    xn = (state - mu) / jnp.sqrt(var + 1e-5)
    xn = xn * lng_ref[...] + lnb_ref[...]
    x = jax.nn.silu(xn)
    x = jax.nn.silu(jnp.dot(x, f1w_ref[...],
                            preferred_element_type=jnp.float32) + f1b_ref[...])
    out = jnp.sum(x * f2w_ref[...], axis=1, keepdims=True) + f2b_ref[0, 0]
    out_ref[...] = jnp.broadcast_to(out, out_ref.shape)


def _head(parts0, parts1, pw0, pb0, pw1, pb1, lng, lnb, f1w, f1b, f2w, f2b):
    b = parts0[0].shape[0]
    hdim = parts0[0].shape[1]
    bb = 256
    grid = (b // bb,)

    def part_spec(width):
        return pl.BlockSpec((bb, width), lambda i: (i, 0))

    def full_spec(shape):
        return pl.BlockSpec(shape, lambda i: tuple(0 for _ in shape))

    in_specs = (
        [part_spec(hdim), part_spec(hdim), part_spec(hdim), part_spec(128)] * 2
        + [full_spec((4 * hdim, hdim)), full_spec((1, hdim)),
           full_spec((4 * hdim, hdim)), full_spec((1, hdim)),
           full_spec((1, 2 * hdim)), full_spec((1, 2 * hdim)),
           full_spec((2 * hdim, hdim)), full_spec((1, hdim)),
           full_spec((1, hdim)), full_spec((1, 128))]
    )
    out2d = pl.pallas_call(
        _head_body,
        grid=grid,
        in_specs=in_specs,
        out_specs=pl.BlockSpec((bb, 128), lambda i: (i, 0)),
        out_shape=jax.ShapeDtypeStruct((b, 128), jnp.float32),
    )(*parts0, *parts1,
      pw0, pb0.reshape(1, hdim), pw1, pb1.reshape(1, hdim),
      lng.reshape(1, 2 * hdim), lnb.reshape(1, 2 * hdim),
      f1w, f1b.reshape(1, hdim), f2w.reshape(1, hdim),
      jnp.broadcast_to(f2b.reshape(1, 1), (1, 128)))
    return out2d[:, 0]


def kernel(feats_0, batch_0, feats_1, batch_1, gate_W_0, gate_b_0, proj_W_0,
           proj_b_0, gate_W_1, gate_b_1, proj_W_1, proj_b_1, ln_g, ln_b,
           fin1_W, fin1_b, fin2_W, fin2_b):
    parts0 = _segment_reduce(feats_0, batch_0, gate_W_0, gate_b_0,
                             NUM_SEGMENTS)
    parts1 = _segment_reduce(feats_1, batch_1, gate_W_1, gate_b_1,
                             NUM_SEGMENTS)
    return _head(parts0, parts1, proj_W_0, proj_b_0, proj_W_1, proj_b_1,
                 ln_g, ln_b, fin1_W, fin1_b, fin2_W, fin2_b)


# speedup vs baseline: 1.1562x; 1.1562x over previous
import functools

import jax
import jax.numpy as jnp
from jax import lax
from jax.experimental import pallas as pl
from jax.experimental.pallas import tpu as pltpu

NUM_SEGMENTS = 2048
R = 640
SB = 8


def _segment_reduce_body(wl_j, wl_chunk, wl_valid,
                         feats_ref, ids_ref, gw_ref, gb_ref,
                         sum_ref, gsum_ref, mx_ref, cnt_ref,
                         *, n_rows):
    t = pl.program_id(0)
    j = wl_j[t]
    chunk = wl_chunk[t]
    valid = wl_valid[t]

    h = feats_ref[...]
    ids = ids_ref[...]

    row_idx = chunk * R + lax.broadcasted_iota(jnp.int32, (R, 1), 0)
    seg_iota = lax.broadcasted_iota(jnp.int32, (1, SB), 1) + j * SB
    oh_bool = (ids == seg_iota) & (row_idx < n_rows) & (valid > 0)
    ohf = oh_bool.astype(jnp.float32)

    glog = jnp.dot(h, gw_ref[...], preferred_element_type=jnp.float32)
    g = jax.nn.sigmoid(glog[:, :1] + gb_ref[0, 0])

    ohcat = jnp.concatenate([ohf, ohf * g], axis=1)
    partial = lax.dot_general(ohcat, h, (((0,), (0,)), ((), ())),
                              preferred_element_type=jnp.float32)
    psum = partial[:SB, :]
    pgsum = partial[SB:, :]
    pcnt = jnp.broadcast_to(jnp.sum(ohf, axis=0).reshape(SB, 1), (SB, 128))

    cols = []
    for s in range(SB):
        masked = jnp.where(oh_bool[:, s:s + 1], h, -jnp.inf)
        cols.append(jnp.max(masked, axis=0, keepdims=True))
    pmax = jnp.concatenate(cols, axis=0)

    prev_j = wl_j[jnp.maximum(t - 1, 0)]
    first = (t == 0) | (j != prev_j)

    @pl.when(first)
    def _():
        sum_ref[...] = psum
        gsum_ref[...] = pgsum
        mx_ref[...] = pmax
        cnt_ref[...] = pcnt

    @pl.when(jnp.logical_not(first))
    def _():
        sum_ref[...] += psum
        gsum_ref[...] += pgsum
        mx_ref[...] = jnp.maximum(mx_ref[...], pmax)
        cnt_ref[...] += pcnt


def _segment_reduce(feats, batch, gate_w, gate_b, num_segments):
    n, hdim = feats.shape
    nb = -(-n // R)
    npad = nb * R - n
    if npad:
        feats = jnp.pad(feats, ((0, npad), (0, 0)))
        batch = jnp.pad(batch, (0, npad))
    batch = batch.astype(jnp.int32)
    ids2 = batch.reshape(nb * R, 1)

    nj = num_segments // SB
    bounds = jnp.arange(0, num_segments + 1, SB, dtype=batch.dtype)
    jstarts = jnp.searchsorted(batch[:n] if npad else batch, bounds,
                               side='left').astype(jnp.int32)
    first_c = jnp.minimum(jstarts[:-1] // R, nb - 1)
    nonempty = jstarts[1:] > jstarts[:-1]
    last_c = jnp.where(nonempty, (jstarts[1:] - 1) // R, first_c)
    nch = last_c - first_c + 1
    off = jnp.concatenate([jnp.zeros((1,), jnp.int32), jnp.cumsum(nch)])
    total = off[-1]

    t_len = nb + nj
    t_idx = jnp.arange(t_len, dtype=jnp.int32)
    jj = jnp.clip(jnp.searchsorted(off, t_idx, side='right') - 1, 0, nj - 1)
    jj = jj.astype(jnp.int32)
    ch = jnp.clip(first_c[jj] + (t_idx - off[jj]), 0, nb - 1).astype(jnp.int32)
    wl_valid = (t_idx < total).astype(jnp.int32)

    gw = jnp.broadcast_to(gate_w.reshape(hdim, 1), (hdim, 128))
    gb = jnp.broadcast_to(gate_b.reshape(1, 1), (1, 128))

    grid_spec = pltpu.PrefetchScalarGridSpec(
        num_scalar_prefetch=3,
        grid=(t_len,),
        in_specs=[
            pl.BlockSpec((R, hdim), lambda t, wj, wc, wv: (wc[t], 0)),
            pl.BlockSpec((R, 1), lambda t, wj, wc, wv: (wc[t], 0)),
            pl.BlockSpec((hdim, 128), lambda t, wj, wc, wv: (0, 0)),
            pl.BlockSpec((1, 128), lambda t, wj, wc, wv: (0, 0)),
        ],
        out_specs=[
            pl.BlockSpec((SB, hdim), lambda t, wj, wc, wv: (wj[t], 0)),
            pl.BlockSpec((SB, hdim), lambda t, wj, wc, wv: (wj[t], 0)),
            pl.BlockSpec((SB, hdim), lambda t, wj, wc, wv: (wj[t], 0)),
            pl.BlockSpec((SB, 128), lambda t, wj, wc, wv: (wj[t], 0)),
        ],
    )
    out_shape = [
        jax.ShapeDtypeStruct((num_segments, hdim), jnp.float32),
        jax.ShapeDtypeStruct((num_segments, hdim), jnp.float32),
        jax.ShapeDtypeStruct((num_segments, hdim), jnp.float32),
        jax.ShapeDtypeStruct((num_segments, 128), jnp.float32),
    ]
    body = functools.partial(_segment_reduce_body, n_rows=n)
    return pl.pallas_call(
        body,
        grid_spec=grid_spec,
        out_shape=out_shape,
        compiler_params=pltpu.CompilerParams(
            dimension_semantics=("arbitrary",)),
    )(jj, ch, wl_valid, feats, ids2, gw, gb)


def _head_body(sum0_ref, gsum0_ref, mx0_ref, cnt0_ref,
               sum1_ref, gsum1_ref, mx1_ref, cnt1_ref,
               pw0_ref, pb0_ref, pw1_ref, pb1_ref,
               lng_ref, lnb_ref, f1w_ref, f1b_ref, f2w_ref, f2b_ref,
               out_ref):
    def rank(sum_ref, gsum_ref, mx_ref, cnt_ref, pw_ref, pb_ref):
        s = sum_ref[...]
        gs = gsum_ref[...]
        cnt = cnt_ref[:, :1]
        mean = s / jnp.maximum(cnt, 1.0)
        mx = jnp.where(cnt > 0, mx_ref[...], 0.0)
        agg = jnp.concatenate([s, mean, mx, gs], axis=1)
        return jnp.dot(agg, pw_ref[...],
                       preferred_element_type=jnp.float32) + pb_ref[...]

    r0 = rank(sum0_ref, gsum0_ref, mx0_ref, cnt0_ref, pw0_ref, pb0_ref)
    r1 = rank(sum1_ref, gsum1_ref, mx1_ref, cnt1_ref, pw1_ref, pb1_ref)
    state = jnp.concatenate([r0, r1], axis=1)
    mu = jnp.mean(state, axis=1, keepdims=True)
    var = jnp.mean((state - mu) ** 2, axis=1, keepdims=True)
    xn = (state - mu) / jnp.sqrt(var + 1e-5)
    xn = xn * lng_ref[...] + lnb_ref[...]
    x = jax.nn.silu(xn)
    x = jax.nn.silu(jnp.dot(x, f1w_ref[...],
                            preferred_element_type=jnp.float32) + f1b_ref[...])
    out = jnp.sum(x * f2w_ref[...], axis=1, keepdims=True) + f2b_ref[0, 0]
    out_ref[...] = jnp.broadcast_to(out, out_ref.shape)


def _head(parts0, parts1, pw0, pb0, pw1, pb1, lng, lnb, f1w, f1b, f2w, f2b):
    b = parts0[0].shape[0]
    hdim = parts0[0].shape[1]
    bb = 256
    grid = (b // bb,)

    def part_spec(width):
        return pl.BlockSpec((bb, width), lambda i: (i, 0))

    def full_spec(shape):
        return pl.BlockSpec(shape, lambda i: tuple(0 for _ in shape))

    in_specs = (
        [part_spec(hdim), part_spec(hdim), part_spec(hdim), part_spec(128)] * 2
        + [full_spec((4 * hdim, hdim)), full_spec((1, hdim)),
           full_spec((4 * hdim, hdim)), full_spec((1, hdim)),
           full_spec((1, 2 * hdim)), full_spec((1, 2 * hdim)),
           full_spec((2 * hdim, hdim)), full_spec((1, hdim)),
           full_spec((1, hdim)), full_spec((1, 128))]
    )
    out2d = pl.pallas_call(
        _head_body,
        grid=grid,
        in_specs=in_specs,
        out_specs=pl.BlockSpec((bb, 128), lambda i: (i, 0)),
        out_shape=jax.ShapeDtypeStruct((b, 128), jnp.float32),
    )(*parts0, *parts1,
      pw0, pb0.reshape(1, hdim), pw1, pb1.reshape(1, hdim),
      lng.reshape(1, 2 * hdim), lnb.reshape(1, 2 * hdim),
      f1w, f1b.reshape(1, hdim), f2w.reshape(1, hdim),
      jnp.broadcast_to(f2b.reshape(1, 1), (1, 128)))
    return out2d[:, 0]


def kernel(feats_0, batch_0, feats_1, batch_1, gate_W_0, gate_b_0, proj_W_0,
           proj_b_0, gate_W_1, gate_b_1, proj_W_1, proj_b_1, ln_g, ln_b,
           fin1_W, fin1_b, fin2_W, fin2_b):
    parts0 = _segment_reduce(feats_0, batch_0, gate_W_0, gate_b_0,
                             NUM_SEGMENTS)
    parts1 = _segment_reduce(feats_1, batch_1, gate_W_1, gate_b_1,
                             NUM_SEGMENTS)
    return _head(parts0, parts1, proj_W_0, proj_b_0, proj_W_1, proj_b_1,
                 ln_g, ln_b, fin1_W, fin1_b, fin2_W, fin2_b)
